# Initial kernel scaffold; baseline (speedup 1.0000x reference)
import functools
import math

import jax
import jax.numpy as jnp
from jax import lax
from jax.experimental import pallas as pl
from jax.experimental.pallas import tpu as pltpu
from jax.experimental.pallas import tpu_sc as plsc

_VOCAB = 1000000
_D = 32
_BATCH = 4096
_SEQ = 200
_SCALE = math.sqrt(_D)

_NC = 2
_NS = 16
_NW = _NC * _NS

_N = _BATCH * _SEQ
_NPW = _N // _NW
_C = 1024
_NCHUNK = _NPW // _C
_UNROLL = 8

_mesh = plsc.VectorSubcoreMesh(core_axis_name="c", subcore_axis_name="s")


@functools.partial(
    pl.kernel,
    mesh=_mesh,
    out_type=jax.ShapeDtypeStruct((_N, _D), jnp.float32),
    scratch_types=[
        pltpu.VMEM((_C,), jnp.int32),
        pltpu.VMEM((_C, _D), jnp.float32),
        pltpu.SemaphoreType.DMA,
    ],
)
def _emb_lookup(x_hbm, tab_hbm, out_hbm, idx_v, rows_v, sem):
    wid = lax.axis_index("s") * _NC + lax.axis_index("c")
    base = wid * _NPW

    def chunk_body(ci, carry):
        off = base + ci * _C
        pltpu.sync_copy(x_hbm.at[pl.ds(off, _C)], idx_v)
        pltpu.async_copy(tab_hbm.at[idx_v], rows_v, sem).wait()

        def scale_body(i, carry2):
            r0 = i * _UNROLL
            for u in range(_UNROLL):
                for h in range(_D // 16):
                    sl = (r0 + u, pl.ds(h * 16, 16))
                    rows_v[sl] = rows_v[sl] * _SCALE
            return carry2

        lax.fori_loop(0, _C // _UNROLL, scale_body, 0)
        pltpu.sync_copy(rows_v, out_hbm.at[pl.ds(off, _C)])
        return carry

    lax.fori_loop(0, _NCHUNK, chunk_body, 0)


def kernel(x, emb_table):
    out = _emb_lookup(x.reshape(_N), emb_table)
    return out.reshape(_BATCH, _SEQ, _D)


# speedup vs baseline: 1.3997x; 1.3997x over previous
import functools
import math

import jax
import jax.numpy as jnp
from jax import lax
from jax.experimental import pallas as pl
from jax.experimental.pallas import tpu as pltpu
from jax.experimental.pallas import tpu_sc as plsc

_VOCAB = 1000000
_D = 32
_BATCH = 4096
_SEQ = 200
_SCALE = math.sqrt(_D)

_NC = 2
_NS = 16
_NW = _NC * _NS

_N = _BATCH * _SEQ
_NPW = _N // _NW
_C = 1024
_NCHUNK = _NPW // _C
_UNROLL = 8

_mesh = plsc.VectorSubcoreMesh(core_axis_name="c", subcore_axis_name="s")


@functools.partial(
    pl.kernel,
    mesh=_mesh,
    out_type=jax.ShapeDtypeStruct((_N, _D), jnp.float32),
    scratch_types=[
        pltpu.VMEM((_C,), jnp.int32),
        pltpu.VMEM((_C, _D), jnp.float32),
        pltpu.SemaphoreType.DMA,
    ],
    compiler_params=pltpu.CompilerParams(use_tc_tiling_on_sc=False),
)
def _emb_lookup(x_hbm, tab_hbm, out_hbm, idx_v, rows_v, sem):
    wid = lax.axis_index("s") * _NC + lax.axis_index("c")
    base = wid * _NPW

    def chunk_body(ci, carry):
        off = base + ci * _C
        pltpu.sync_copy(x_hbm.at[pl.ds(off, _C)], idx_v)
        pltpu.async_copy(tab_hbm.at[idx_v], rows_v, sem).wait()

        def scale_body(i, carry2):
            r0 = i * _UNROLL
            for u in range(_UNROLL):
                for h in range(_D // 16):
                    sl = (r0 + u, pl.ds(h * 16, 16))
                    rows_v[sl] = rows_v[sl] * _SCALE
            return carry2

        lax.fori_loop(0, _C // _UNROLL, scale_body, 0)
        pltpu.sync_copy(rows_v, out_hbm.at[pl.ds(off, _C)])
        return carry

    lax.fori_loop(0, _NCHUNK, chunk_body, 0)


def kernel(x, emb_table):
    out = _emb_lookup(x.reshape(_N), emb_table)
    return out.reshape(_BATCH, _SEQ, _D)
